# baseline (speedup 1.0000x reference)
import functools

import jax
import jax.numpy as jnp
from jax import lax
from jax.experimental import pallas as pl
from jax.experimental.pallas import tpu as pltpu
from jax.experimental.pallas import tpu_sc as plsc

N = 1048576
NC = 2
NS = 16
L = 16
NW = NC * NS
R = N // NW
C = 4096
NCHUNK = R // C


def _huber16(p, t):
    a = jnp.abs(p - t)
    q = jnp.minimum(a, 1.0)
    return 0.5 * q * q + (a - q)


def _sc_loss(pred_flat, labels, targ_flat):
    mesh = plsc.VectorSubcoreMesh(
        core_axis_name="c", subcore_axis_name="s",
        num_cores=NC, num_subcores=NS,
    )

    @functools.partial(
        pl.kernel,
        mesh=mesh,
        out_type=(
            jax.ShapeDtypeStruct((NW, L), jnp.float32),
            jax.ShapeDtypeStruct((NW, L), jnp.float32),
        ),
        scratch_types=[
            pltpu.VMEM((4 * C,), jnp.float32),
            pltpu.VMEM((4 * C,), jnp.float32),
            pltpu.VMEM((C,), jnp.int32),
            pltpu.VMEM((4 * C,), jnp.float32),
            pltpu.VMEM((4 * C,), jnp.float32),
            pltpu.VMEM((C,), jnp.int32),
            pltpu.VMEM((L,), jnp.float32),
            pltpu.VMEM((L,), jnp.float32),
            pltpu.SemaphoreType.DMA,
            pltpu.SemaphoreType.DMA,
        ],
    )
    def k(pred_hbm, lab_hbm, targ_hbm, acc_hbm, cnt_hbm,
          p0, t0, l0, p1, t1, l1, acc_v, cnt_v, sem0, sem1):
        cid = lax.axis_index("c")
        sid = lax.axis_index("s")
        wid = sid * NC + cid
        base = wid * R

        bufs = ((p0, t0, l0), (p1, t1, l1))
        sems = (sem0, sem1)

        def start(ci, bi):
            pb, tb, lb = bufs[bi]
            r0 = base + ci * C
            return (
                pltpu.async_copy(pred_hbm.at[pl.ds(4 * r0, 4 * C)], pb, sems[bi]),
                pltpu.async_copy(targ_hbm.at[pl.ds(4 * r0, 4 * C)], tb, sems[bi]),
                pltpu.async_copy(lab_hbm.at[pl.ds(r0, C)], lb, sems[bi]),
            )

        lane = lax.iota(jnp.int32, L)
        rvec = lax.shift_right_logical(lane, 1)
        rvec = lax.shift_right_logical(rvec, 1)

        def chunk(bi, acc, cnt):
            pb, tb, lb = bufs[bi]

            def body(i, carry):
                a, c = carry
                lab = lb[pl.ds(i * L, L)]
                m = jnp.where(lab == 1, 1.0, 0.0).astype(jnp.float32)
                c = c + m
                for j in range(4):
                    mj = jnp.take_along_axis(
                        m, 4 * j + rvec, axis=0, mode="promise_in_bounds")
                    p = pb[pl.ds(i * 4 * L + j * L, L)]
                    t = tb[pl.ds(i * 4 * L + j * L, L)]
                    a = a + mj * _huber16(p, t)
                return a, c

            return lax.fori_loop(0, C // L, body, (acc, cnt))

        acc = jnp.zeros((L,), jnp.float32)
        cnt = jnp.zeros((L,), jnp.float32)
        cps = start(0, 0)
        for ci in range(NCHUNK):
            nxt = start(ci + 1, (ci + 1) % 2) if ci + 1 < NCHUNK else None
            for cp in cps:
                cp.wait()
            acc, cnt = chunk(ci % 2, acc, cnt)
            cps = nxt
        acc_v[...] = acc
        cnt_v[...] = cnt
        pltpu.sync_copy(acc_v, acc_hbm.at[wid])
        pltpu.sync_copy(cnt_v, cnt_hbm.at[wid])

    return k(pred_flat, labels, targ_flat)


def kernel(out_bbox, labels, bbox_targets):
    pred = out_bbox.reshape(-1)
    targ = bbox_targets.reshape(-1)
    acc, cnt = _sc_loss(pred, labels, targ)
    return jnp.sum(acc) * 0.25 / jnp.sum(cnt)
